# baseline (speedup 1.0000x reference)
import functools

import jax
import jax.numpy as jnp
from jax import lax
from jax.experimental import pallas as pl
from jax.experimental.pallas import tpu as pltpu
from jax.experimental.pallas import tpu_sc as plsc

D = 256
LANES = 16
CHUNK = 40


def _norm_body(z_ref, o_ref):
    x = z_ref[...]
    ss = jnp.sum(x * x, axis=1, keepdims=True)
    nrm = jnp.sqrt(ss)
    o_ref[...] = (x / jnp.maximum(nrm, 1e-12)).astype(jnp.bfloat16)


def _normalize(z):
    n, d = z.shape
    blk = 2000
    return pl.pallas_call(
        _norm_body,
        out_shape=jax.ShapeDtypeStruct((n, d), jnp.bfloat16),
        grid=(n // blk,),
        in_specs=[pl.BlockSpec((blk, d), lambda i: (i, 0))],
        out_specs=pl.BlockSpec((blk, d), lambda i: (i, 0)),
    )(z)


def _w_body(zn_ref, rel_ref, o_ref):
    zb = zn_ref[...]
    rf = rel_ref[...]
    p = jax.lax.dot_general(
        zb, rf.astype(jnp.bfloat16),
        (((1,), (1,)), ((), ())),
        preferred_element_type=jnp.float32)
    rbias = 0.5 * jnp.sum(rf * rf, axis=1) + 1.0
    o_ref[...] = (2.0 * p - rbias[None, :]).astype(jnp.bfloat16)


def _w_table(zn_bf, rel_f32):
    n = zn_bf.shape[0]
    r = rel_f32.shape[0]
    blk = 2000
    return pl.pallas_call(
        _w_body,
        out_shape=jax.ShapeDtypeStruct((n, r), jnp.bfloat16),
        grid=(n // blk,),
        in_specs=[
            pl.BlockSpec((blk, D), lambda i: (i, 0)),
            pl.BlockSpec((r, D), lambda i: (0, 0)),
        ],
        out_specs=pl.BlockSpec((blk, r), lambda i: (i, 0)),
    )(zn_bf, rel_f32)


@functools.cache
def _make_sc_scorer(num_edges, rhalf):
    info = plsc.get_sparse_core_info()
    nc, ns = info.num_cores, info.num_subcores
    nw = nc * ns
    epw = num_edges // nw
    assert epw * nw == num_edges and epw % CHUNK == 0
    nchunks = epw // CHUNK
    ngr = -(-CHUNK // LANES)

    mesh = plsc.VectorSubcoreMesh(core_axis_name="c", subcore_axis_name="s")

    @functools.partial(
        pl.kernel,
        mesh=mesh,
        out_type=jax.ShapeDtypeStruct((num_edges,), jnp.float32),
        compiler_params=pltpu.CompilerParams(
            use_tc_tiling_on_sc=False, needs_layout_passes=False),
        scratch_types=[
            pltpu.VMEM((epw + LANES,), jnp.int32),
            pltpu.VMEM((epw + LANES,), jnp.int32),
            pltpu.VMEM((epw + LANES,), jnp.int32),
            pltpu.VMEM((epw,), jnp.float32),
            pltpu.VMEM((CHUNK, D), jnp.bfloat16),
            pltpu.VMEM((CHUNK, D), jnp.bfloat16),
            pltpu.VMEM((CHUNK, D), jnp.bfloat16),
            pltpu.VMEM((CHUNK, D), jnp.bfloat16),
            pltpu.VMEM((ngr * LANES,), jnp.int32),
            pltpu.VMEM((ngr * LANES,), jnp.int32),
            pltpu.VMEM((ngr * LANES,), jnp.int32),
            pltpu.VMEM((ngr * LANES,), jnp.int32),
            pltpu.VMEM((CHUNK + 8,), jnp.int32),
            pltpu.VMEM((CHUNK + 8,), jnp.int32),
            pltpu.VMEM((CHUNK + 8,), jnp.int32),
            pltpu.VMEM((CHUNK + 8,), jnp.int32),
            pltpu.SemaphoreType.DMA,
            pltpu.SemaphoreType.DMA,
        ],
    )
    def scorer(zn_hbm, wpk_hbm, ei_hbm, rt_hbm, out_hbm,
               src_v, dst_v, rt_v, out_v,
               srows0, drows0, srows1, drows1,
               wsx0, wdx0, wsx1, wdx1,
               ws0, wd0, ws1, wd1, s0, s1):
        wid = lax.axis_index("s") * nc + lax.axis_index("c")
        base = pl.multiple_of(wid * epw, 8)
        pltpu.sync_copy(ei_hbm.at[0, pl.ds(base, epw)],
                        src_v.at[pl.ds(0, epw)])
        pltpu.sync_copy(ei_hbm.at[1, pl.ds(base, epw)],
                        dst_v.at[pl.ds(0, epw)])
        pltpu.sync_copy(rt_hbm.at[pl.ds(base, epw)], rt_v.at[pl.ds(0, epw)])

        lanes = lax.iota(jnp.int32, LANES)
        bufs = ((srows0, drows0, wsx0, wdx0, ws0, wd0, s0),
                (srows1, drows1, wsx1, wdx1, ws1, wd1, s1))
        zero = jnp.zeros((LANES,), jnp.float32)

        def _descs(c, b):
            off = pl.multiple_of(c * CHUNK, 8)
            sr, dr, wsx, wdx, ws, wd, sem = bufs[b]
            return (
                pltpu.make_async_copy(
                    zn_hbm.at[src_v.at[pl.ds(off, CHUNK)]], sr, sem),
                pltpu.make_async_copy(
                    zn_hbm.at[dst_v.at[pl.ds(off, CHUNK)]], dr, sem),
                pltpu.make_async_copy(
                    wpk_hbm.at[wsx.at[pl.ds(0, CHUNK)]],
                    ws.at[pl.ds(0, CHUNK)], sem),
                pltpu.make_async_copy(
                    wpk_hbm.at[wdx.at[pl.ds(0, CHUNK)]],
                    wd.at[pl.ds(0, CHUNK)], sem),
            )

        def _fire(c, b):
            off = pl.multiple_of(c * CHUNK, 8)
            sr, dr, wsx, wdx, ws, wd, sem = bufs[b]
            for h in range(ngr):
                o = pl.multiple_of(off + h * LANES, 8)
                rv = rt_v[pl.ds(o, LANES)] >> 1
                wsx[pl.ds(h * LANES, LANES)] = (
                    src_v[pl.ds(o, LANES)] * rhalf + rv)
                wdx[pl.ds(h * LANES, LANES)] = (
                    dst_v[pl.ds(o, LANES)] * rhalf + rv)
            for h in _descs(c, b):
                h.start()

        def _wait(c, b):
            for h in _descs(c, b):
                h.wait()

        def _unpack2(v):
            return plsc.unpack(
                v, format=plsc.PackFormat.INTERLEAVED,
                preferred_element_type=jnp.float32)

        def _compute(c, b):
            off = pl.multiple_of(c * CHUNK, 8)
            sr, dr, wsx, wdx, ws, wd, _ = bufs[b]
            for g in range(ngr):
                nv = min(LANES, CHUNK - g * LANES)

                def e_body(k, dvec):
                    e = g * LANES + k
                    acc0 = zero
                    acc1 = zero
                    for j in range(0, D // 32, 2):
                        qq = None
                        for jj in (j, j + 1):
                            vs = sr[e, pl.ds(jj * 32, 32)]
                            vd = dr[e, pl.ds(jj * 32, 32)]
                            q = vs * vd
                            qq = q if qq is None else qq + q
                        q0, q1 = _unpack2(qq)
                        acc0 = acc0 + q0
                        acc1 = acc1 + q1
                    dot = jnp.sum(acc0 + acc1)
                    return jnp.where(lanes == k, dot, dvec)

                dvec = lax.fori_loop(0, nv, e_body, zero, unroll=2)
                even = (rt_v[pl.ds(off + g * LANES, LANES)] & 1) == 0
                l1, h1 = _unpack2(
                    plsc.bitcast(ws[pl.ds(g * LANES, LANES)], jnp.bfloat16))
                l2, h2 = _unpack2(
                    plsc.bitcast(wd[pl.ds(g * LANES, LANES)], jnp.bfloat16))
                w1 = jnp.where(even, l1, h1)
                w2 = jnp.where(even, l2, h2)
                svec = (w1 + w2) - (dvec + dvec)
                if nv == LANES:
                    out_v[pl.ds(off + g * LANES, LANES)] = svec
                else:
                    plsc.store_scatter(
                        out_v, [off + g * LANES + lanes], svec,
                        mask=lanes < nv)

        _fire(0, 0)

        def pair_body(i, carry):
            c = pl.multiple_of(i * 2, 2)
            _fire(c + 1, 1)
            _wait(c, 0)
            _compute(c, 0)
            _fire(c + 2, 0)
            _wait(c + 1, 1)
            _compute(c + 1, 1)
            return carry

        lax.fori_loop(0, (nchunks - 1) // 2, pair_body, 0)
        _wait(nchunks - 1, 0)
        _compute(nchunks - 1, 0)
        pltpu.sync_copy(out_v, out_hbm.at[pl.ds(base, epw)])

    return scorer


def kernel(z, edge_index, edge_type, rel_emb):
    r = rel_emb.shape[0]
    n = z.shape[0]
    zn = _normalize(z)
    wbf = _w_table(zn, rel_emb)
    wpk = jax.lax.bitcast_convert_type(
        wbf.reshape(n * r // 2, 2), jnp.int32)
    ei = edge_index.astype(jnp.int32)
    rt = edge_type.astype(jnp.int32)
    scorer = _make_sc_scorer(edge_type.shape[0], r // 2)
    return scorer(zn, wpk, ei, rt)


# speedup vs baseline: 6.9078x; 6.9078x over previous
import functools

import jax
import jax.numpy as jnp
from jax import lax
from jax.experimental import pallas as pl
from jax.experimental.pallas import tpu as pltpu
from jax.experimental.pallas import tpu_sc as plsc

D = 256
LANES = 16
CHUNK = 40


def _norm_body(z_ref, o_ref):
    x = z_ref[...]
    ss = jnp.sum(x * x, axis=1, keepdims=True)
    nrm = jnp.sqrt(ss)
    o_ref[...] = (x / jnp.maximum(nrm, 1e-12)).astype(jnp.bfloat16)


def _normalize(z):
    n, d = z.shape
    blk = 2000
    return pl.pallas_call(
        _norm_body,
        out_shape=jax.ShapeDtypeStruct((n, d), jnp.bfloat16),
        grid=(n // blk,),
        in_specs=[pl.BlockSpec((blk, d), lambda i: (i, 0))],
        out_specs=pl.BlockSpec((blk, d), lambda i: (i, 0)),
    )(z)


@functools.cache
def _make_sc_scorer(num_edges, num_rel):
    info = plsc.get_sparse_core_info()
    nc, ns = info.num_cores, info.num_subcores
    nw = nc * ns
    epw = num_edges // nw
    assert epw * nw == num_edges and epw % CHUNK == 0
    nchunks = epw // CHUNK

    mesh = plsc.VectorSubcoreMesh(core_axis_name="c", subcore_axis_name="s")

    @functools.partial(
        pl.kernel,
        mesh=mesh,
        out_type=jax.ShapeDtypeStruct((num_edges,), jnp.float32),
        compiler_params=pltpu.CompilerParams(
            use_tc_tiling_on_sc=False, needs_layout_passes=False),
        scratch_types=[
            pltpu.VMEM((epw,), jnp.int32),
            pltpu.VMEM((epw,), jnp.int32),
            pltpu.VMEM((epw + LANES,), jnp.int32),
            pltpu.VMEM((epw,), jnp.float32),
            pltpu.VMEM((num_rel, D // 2), jnp.int32),
            pltpu.VMEM((CHUNK, D), jnp.bfloat16),
            pltpu.VMEM((CHUNK, D), jnp.bfloat16),
            pltpu.VMEM((CHUNK, D), jnp.bfloat16),
            pltpu.VMEM((CHUNK, D), jnp.bfloat16),
            pltpu.SemaphoreType.DMA,
            pltpu.SemaphoreType.DMA,
            pltpu.SemaphoreType.DMA,
        ],
    )
    def scorer(zn_hbm, ei_hbm, rt_hbm, rel_hbm, out_hbm,
               src_v, dst_v, rt_v, out_v, relt,
               srows0, drows0, srows1, drows1, s0, s1, srel):
        wid = lax.axis_index("s") * nc + lax.axis_index("c")
        base = pl.multiple_of(wid * epw, 8)
        hrel = pltpu.make_async_copy(rel_hbm, relt, srel)
        hrel.start()
        pltpu.sync_copy(ei_hbm.at[0, pl.ds(base, epw)], src_v)
        pltpu.sync_copy(ei_hbm.at[1, pl.ds(base, epw)], dst_v)
        pltpu.sync_copy(rt_hbm.at[pl.ds(base, epw)], rt_v.at[pl.ds(0, epw)])

        lanes = lax.iota(jnp.int32, LANES)
        bufs = ((srows0, drows0, s0), (srows1, drows1, s1))
        zero = jnp.zeros((LANES,), jnp.float32)

        def _descs(c, b):
            off = pl.multiple_of(c * CHUNK, 8)
            sr, dr, sem = bufs[b]
            return (
                pltpu.make_async_copy(
                    zn_hbm.at[src_v.at[pl.ds(off, CHUNK)]], sr, sem),
                pltpu.make_async_copy(
                    zn_hbm.at[dst_v.at[pl.ds(off, CHUNK)]], dr, sem),
            )

        def _fire(c, b):
            for h in _descs(c, b):
                h.start()

        def _wait(c, b):
            for h in _descs(c, b):
                h.wait()

        def _unpack2(v):
            return plsc.unpack(
                v, format=plsc.PackFormat.INTERLEAVED,
                preferred_element_type=jnp.float32)

        take_dnums = lax.GatherDimensionNumbers(
            offset_dims=(), collapsed_slice_dims=(0,), start_index_map=(0,))

        def _bcast_lane(vec, k):
            idx = jnp.full((LANES, 1), k, dtype=jnp.int32)
            return lax.gather(
                vec, idx, take_dnums, (1,),
                mode=lax.GatherScatterMode.PROMISE_IN_BOUNDS)

        cols = tuple(j * LANES + lanes for j in range(D // 32))

        def _compute(c, b):
            off = pl.multiple_of(c * CHUNK, 8)
            sr, dr, _ = bufs[b]
            for g in range(-(-CHUNK // LANES)):
                nv = min(LANES, CHUNK - g * LANES)
                te_vec = rt_v[pl.ds(off + g * LANES, LANES)]

                def e_body(k, svec):
                    e = g * LANES + k
                    te = _bcast_lane(te_vec, k)
                    acc0 = zero
                    acc1 = zero
                    for j in range(0, D // 32, 2):
                        qq = None
                        for jj in (j, j + 1):
                            vs = sr[e, pl.ds(jj * 32, 32)]
                            vd = dr[e, pl.ds(jj * 32, 32)]
                            vr = plsc.bitcast(
                                plsc.load_gather(relt, [te, cols[jj]]),
                                jnp.bfloat16)
                            t = (vs + vd) - vr
                            q = t * t
                            qq = q if qq is None else qq + q
                        q0, q1 = _unpack2(qq)
                        acc0 = acc0 + q0
                        acc1 = acc1 + q1
                    total = jnp.sum(acc0 + acc1)
                    return jnp.where(lanes == k, -total, svec)

                svec = lax.fori_loop(0, nv, e_body, zero, unroll=2)
                if nv == LANES:
                    out_v[pl.ds(off + g * LANES, LANES)] = svec
                else:
                    plsc.store_scatter(
                        out_v, [off + g * LANES + lanes], svec,
                        mask=lanes < nv)

        _fire(0, 0)
        hrel.wait()

        def pair_body(i, carry):
            c = pl.multiple_of(i * 2, 2)
            _fire(c + 1, 1)
            _wait(c, 0)
            _compute(c, 0)
            _fire(c + 2, 0)
            _wait(c + 1, 1)
            _compute(c + 1, 1)
            return carry

        lax.fori_loop(0, (nchunks - 1) // 2, pair_body, 0)
        _wait(nchunks - 1, 0)
        _compute(nchunks - 1, 0)
        pltpu.sync_copy(out_v, out_hbm.at[pl.ds(base, epw)])

    return scorer


def kernel(z, edge_index, edge_type, rel_emb):
    r = rel_emb.shape[0]
    zn = _normalize(z)
    rel_i = jax.lax.bitcast_convert_type(
        rel_emb.astype(jnp.bfloat16).reshape(r, D // 2, 2), jnp.int32)
    ei = edge_index.astype(jnp.int32)
    rt = edge_type.astype(jnp.int32)
    scorer = _make_sc_scorer(edge_type.shape[0], r)
    return scorer(zn, ei, rt, rel_i)
